# baseline (speedup 1.0000x reference)
import functools

import jax
import jax.numpy as jnp
import numpy as np
from jax.experimental import pallas as pl

N_NODES = 50000
N_GRAPHS = 512
D = 64
R = 20
CUTOFF = 5.0
L = 3


def _out_body(h_ref, x_ref, w1_ref, w2_ref, o_ref):
    h = h_ref[...]
    s = jax.nn.silu(h @ w1_ref[...]) @ w2_ref[...]
    x = x_ref[...]
    o_ref[...] = jnp.sum(x * s[None, :, :], axis=-1).T


def _final_out(h, X3, W_out1, W_out2):
    n = h.shape[0]
    B = 1000
    grid = (n // B,)
    return pl.pallas_call(
        _out_body,
        grid=grid,
        in_specs=[
            pl.BlockSpec((B, D), lambda i: (i, 0)),
            pl.BlockSpec((3, B, D), lambda i: (0, i, 0)),
            pl.BlockSpec((D, D), lambda i: (0, 0)),
            pl.BlockSpec((D, D), lambda i: (0, 0)),
        ],
        out_specs=pl.BlockSpec((B, 3), lambda i: (i, 0)),
        out_shape=jax.ShapeDtypeStruct((n, 3), jnp.float32),
    )(h, X3, W_out1, W_out2)


def kernel(ts_coord_t, numbers_t, bmat_r_t, bmat_p_t, edge_index, batch, time,
           atom_emb, edge_emb, W_time, W_rfeat, W_pfeat,
           W_msg, W_rbf, W_upd, W_vec, W_out1, W_out2):
    N = ts_coord_t.shape[0]
    edge_type = bmat_r_t * 22 + bmat_p_t
    src = edge_index[0]
    dst = edge_index[1]
    mask = (src != dst) & (edge_type != 0) & (batch[src] == batch[dst])
    maskf = mask.astype(jnp.float32)[:, None]

    vec = ts_coord_t[dst] - ts_coord_t[src]
    dist = jnp.sqrt(jnp.sum(vec * vec, axis=-1) + 1e-12)
    dirv = vec / dist[:, None]
    means = jnp.linspace(float(np.exp(-CUTOFF)), 1.0, R).astype(jnp.float32)
    beta = float(((2.0 / R) * (1.0 - np.exp(-CUTOFF))) ** -2)
    rbf = jnp.exp(-beta * (jnp.exp(-dist)[:, None] - means[None, :]) ** 2)
    fcut = 0.5 * (jnp.cos(jnp.pi * dist / CUTOFF) + 1.0) * (dist < CUTOFF)
    rbfc = rbf * (fcut * maskf[:, 0])[:, None]

    h = atom_emb[numbers_t] + time[batch][:, None] * W_time[0][None, :]
    e_emb = edge_emb[edge_type] * maskf
    X = jnp.zeros((3, N, D), dtype=jnp.float32)

    for l in range(L):
        hm = jax.nn.silu(h @ W_msg[l])
        filt = rbfc @ W_rbf[l] + e_emb
        m = hm[src] * filt
        agg = jnp.zeros((N, D), jnp.float32).at[dst].add(m)
        h = h + jax.nn.silu(agg @ W_upd[l])
        Y = jnp.zeros((3, N, D), jnp.float32).at[:, dst].add(dirv.T[:, :, None] * m[None, :, :])
        X = X + Y @ W_vec[l]

    return _final_out(h, X, W_out1, W_out2)


# speedup vs baseline: 8.8665x; 8.8665x over previous
import functools

import jax
import jax.numpy as jnp
import numpy as np
from jax import lax
from jax.experimental import pallas as pl
from jax.experimental.pallas import tpu as pltpu
from jax.experimental.pallas import tpu_sc as plsc

NC, NS = 2, 16
NW = NC * NS
CH = 128
D = 64
RPAD = 32
RACC = 16
PW = 16
CUTOFF = 5.0
R = 20
L = 3

_MESH = dict(core_axis_name="c", subcore_axis_name="s")


def _it16():
    return lax.iota(jnp.int32, 16)


def _wid():
    return lax.axis_index("s") * NC + lax.axis_index("c")


def _k_nodeprep(NPAD, n_ch):
    npw = n_ch * CH

    def body(num_ref, bat_ref, coordf_ref, batchf_ref, ae_ref, t_ref,
             aeg_ref, tgg_ref, p_ref, nidx, bidx, rows_a, rows_t, cbuf, bbuf,
             pbuf, sem):
        wid = _wid()

        def chunk(ci, carry):
            base = wid * npw + ci * CH
            pltpu.sync_copy(num_ref.at[pl.ds(base, CH)], nidx)
            pltpu.async_copy(ae_ref.at[nidx], rows_a, sem).wait()
            pltpu.sync_copy(rows_a, aeg_ref.at[pl.ds(base, CH), :])
            pltpu.sync_copy(bat_ref.at[pl.ds(base, CH)], bidx)
            pltpu.async_copy(t_ref.at[bidx], rows_t, sem).wait()
            pltpu.sync_copy(rows_t, tgg_ref.at[pl.ds(base, CH), :])
            pltpu.sync_copy(coordf_ref.at[pl.ds(base * 3, CH * 3)], cbuf)
            pltpu.sync_copy(batchf_ref.at[pl.ds(base, CH)], bbuf)
            it = _it16()
            for g in range(CH // 16):
                j16 = g * 16 + it
                for f in range(3):
                    vals = plsc.load_gather(cbuf, [j16 * 3 + f])
                    plsc.store_scatter(pbuf, [j16 * PW + f], vals)
                bv = bbuf[pl.ds(g * 16, 16)]
                plsc.store_scatter(pbuf, [j16 * PW + 3], bv)
            pltpu.sync_copy(pbuf, p_ref.at[pl.ds(base * PW, CH * PW)])
            return carry

        lax.fori_loop(0, n_ch, chunk, 0)

    return pl.kernel(
        body,
        out_type=(jax.ShapeDtypeStruct((NPAD, D), jnp.float32),
                  jax.ShapeDtypeStruct((NPAD, D), jnp.float32),
                  jax.ShapeDtypeStruct((NPAD * PW,), jnp.float32)),
        mesh=plsc.VectorSubcoreMesh(**_MESH),
        compiler_params=pltpu.CompilerParams(needs_layout_passes=False, use_tc_tiling_on_sc=False),
        scratch_types=[pltpu.VMEM((CH,), jnp.int32),
                       pltpu.VMEM((CH,), jnp.int32),
                       pltpu.VMEM((CH, D), jnp.float32),
                       pltpu.VMEM((CH, D), jnp.float32),
                       pltpu.VMEM((CH * 3,), jnp.float32),
                       pltpu.VMEM((CH,), jnp.float32),
                       pltpu.VMEM((CH * PW,), jnp.float32),
                       pltpu.SemaphoreType.DMA])


def _k_edgeprep(NPAD, EPAD, e_ch):
    epw = e_ch * CH

    def body(src_ref, dst_ref, bmr_ref, bmp_ref, p_ref, ee_ref,
             v4_ref, eeg_ref, sidx, didx, rb, pb2, etb, ps, pd, vbuf, erows,
             sem):
        wid = _wid()

        def chunk(ci, carry):
            base = wid * epw + ci * CH
            pltpu.sync_copy(src_ref.at[pl.ds(base, CH)], sidx)
            pltpu.sync_copy(dst_ref.at[pl.ds(base, CH)], didx)
            pltpu.sync_copy(bmr_ref.at[pl.ds(base, CH)], rb)
            pltpu.sync_copy(bmp_ref.at[pl.ds(base, CH)], pb2)
            pltpu.async_copy(p_ref.at[sidx], ps, sem).wait()
            pltpu.async_copy(p_ref.at[didx], pd, sem).wait()
            for g in range(CH // 16):
                sl = pl.ds(g * 16, 16)
                etb[sl] = rb[sl] * 22 + pb2[sl]
            pltpu.async_copy(ee_ref.at[etb], erows, sem).wait()
            pltpu.sync_copy(erows, eeg_ref.at[pl.ds(base, CH), :])
            it = _it16()
            for g in range(CH // 16):
                j16 = g * 16 + it
                for f in range(3):
                    fv = jnp.full((16,), f, jnp.int32)
                    dvf = plsc.load_gather(pd, [j16, fv])
                    svf = plsc.load_gather(ps, [j16, fv])
                    plsc.store_scatter(vbuf, [j16 * 4 + f], dvf - svf)
                c3 = jnp.full((16,), 3, jnp.int32)
                bs = plsc.load_gather(ps, [j16, c3])
                bd = plsc.load_gather(pd, [j16, c3])
                sl = pl.ds(g * 16, 16)
                sv = sidx[sl]
                dv = didx[sl]
                ev = etb[sl]
                mok = jnp.where((ev != 0) & (sv != dv) & (bs == bd),
                                jnp.float32(1.0), jnp.float32(0.0))
                plsc.store_scatter(vbuf, [j16 * 4 + 3], mok)
            pltpu.sync_copy(vbuf, v4_ref.at[pl.ds(base * 4, CH * 4)])
            return carry

        lax.fori_loop(0, e_ch, chunk, 0)

    return pl.kernel(
        body,
        out_type=(jax.ShapeDtypeStruct((EPAD * 4,), jnp.float32),
                  jax.ShapeDtypeStruct((EPAD, D), jnp.float32)),
        mesh=plsc.VectorSubcoreMesh(**_MESH),
        compiler_params=pltpu.CompilerParams(needs_layout_passes=False, use_tc_tiling_on_sc=False),
        scratch_types=[pltpu.VMEM((CH,), jnp.int32),
                       pltpu.VMEM((CH,), jnp.int32),
                       pltpu.VMEM((CH,), jnp.int32),
                       pltpu.VMEM((CH,), jnp.int32),
                       pltpu.VMEM((CH,), jnp.int32),
                       pltpu.VMEM((CH, PW), jnp.float32),
                       pltpu.VMEM((CH, PW), jnp.float32),
                       pltpu.VMEM((CH * 4,), jnp.float32),
                       pltpu.VMEM((CH, D), jnp.float32),
                       pltpu.SemaphoreType.DMA])


def _k_rowgather(NPAD, EPAD, e_ch, NB=7):
    epw = e_ch * CH
    groups = e_ch // NB

    def body(tab_ref, idx2_ref, out_ref, *rest):
        idxv = rest[0]
        rows = rest[1:1 + NB]
        gsem = rest[1 + NB]
        wsem = rest[2 + NB]
        wid = _wid()
        pltpu.sync_copy(idx2_ref.at[pl.ds(wid * e_ch, e_ch), :], idxv)

        def grp(g0, carry):
            ds = []
            for b in range(NB):
                i = g0 * NB + b
                ds.append(pltpu.async_copy(tab_ref.at[idxv.at[i]], rows[b],
                                           gsem))
            for dsc in ds:
                dsc.wait()
            ws = []
            for b in range(NB):
                i = g0 * NB + b
                ws.append(pltpu.async_copy(
                    rows[b], out_ref.at[pl.ds(wid * epw + i * CH, CH), :],
                    wsem))
            for dsc in ws:
                dsc.wait()
            return carry

        lax.fori_loop(0, groups, grp, 0)

    return pl.kernel(
        body,
        out_type=jax.ShapeDtypeStruct((EPAD, D), jnp.float32),
        mesh=plsc.VectorSubcoreMesh(**_MESH),
        compiler_params=pltpu.CompilerParams(needs_layout_passes=False, use_tc_tiling_on_sc=False),
        scratch_types=[pltpu.VMEM((e_ch, CH), jnp.int32)]
        + [pltpu.VMEM((CH, D), jnp.float32) for _ in range(NB)]
        + [pltpu.SemaphoreType.DMA, pltpu.SemaphoreType.DMA])


def _k_scatter(NPAD, EPAD, e_ch, NB=7):
    e_ch2 = (EPAD // CH) // NS
    groups = e_ch2 // NB
    npw_s = NPAD // NS
    nzch = npw_s // CH

    def body(m8_ref, idx2_ref, zrow_ref, ag_ref, *rest):
        idxv = rest[0]
        acc = rest[1]
        zbuf = rest[2]
        mb = rest[3:3 + NB]
        rsem = rest[3 + NB]
        ssem = rest[4 + NB]
        c = lax.axis_index("c")
        s = lax.axis_index("s")
        pltpu.sync_copy(idx2_ref.at[pl.ds(s * e_ch2, e_ch2), :], idxv)
        pltpu.sync_copy(zrow_ref, zbuf)
        for cc_l in range(8):
            half, ch = divmod(cc_l, 4)
            cc = (c * 2 + half) * 4 + ch

            def zc(z, carry):
                pltpu.sync_copy(zbuf, acc.at[pl.ds(s * npw_s + z * CH, CH), :])
                return carry

            lax.fori_loop(0, nzch, zc, 0)
            plsc.subcore_barrier()

            def grp(g0, carry):
                ds = []
                for b in range(NB):
                    i = g0 * NB + b
                    ds.append(pltpu.async_copy(
                        m8_ref.at[cc, pl.ds((s * e_ch2 + i) * CH, CH), :],
                        mb[b], rsem))
                for dsc in ds:
                    dsc.wait()
                for b in range(NB):
                    i = g0 * NB + b
                    pltpu.sync_copy(mb[b], acc.at[idxv.at[i]], add=True)
                return carry

            lax.fori_loop(0, groups, grp, 0)
            plsc.subcore_barrier()

            def wb(z, carry):
                r0 = s * npw_s + z * CH
                pltpu.sync_copy(acc.at[pl.ds(r0, CH), :],
                                ag_ref.at[cc, pl.ds(r0, CH), :])
                return carry

            lax.fori_loop(0, nzch, wb, 0)
            plsc.subcore_barrier()

    return pl.kernel(
        body,
        out_type=jax.ShapeDtypeStruct((16, NPAD, RACC), jnp.float32),
        mesh=plsc.VectorSubcoreMesh(**_MESH),
        compiler_params=pltpu.CompilerParams(needs_layout_passes=False, use_tc_tiling_on_sc=False),
        scratch_types=[pltpu.VMEM((e_ch2, CH), jnp.int32),
                       pltpu.VMEM_SHARED((NPAD, RACC), jnp.float32),
                       pltpu.VMEM((CH, RACC), jnp.float32)]
        + [pltpu.VMEM((CH, RACC), jnp.float32) for _ in range(NB)]
        + [pltpu.SemaphoreType.DMA, pltpu.SemaphoreType.DMA])


def _h0_body(ae_ref, tg_ref, wm_ref, h_ref, hm_ref):
    h = ae_ref[...] + tg_ref[...]
    h_ref[...] = h
    hm_ref[...] = jax.nn.silu(h @ wm_ref[...])


def _tc_h0(ae_g, tg_g, Wm0, NPAD, B=1024):
    return pl.pallas_call(
        _h0_body,
        grid=(NPAD // B,),
        in_specs=[pl.BlockSpec((B, D), lambda i: (i, 0)),
                  pl.BlockSpec((B, D), lambda i: (i, 0)),
                  pl.BlockSpec((D, D), lambda i: (0, 0))],
        out_specs=[pl.BlockSpec((B, D), lambda i: (i, 0)),
                   pl.BlockSpec((B, D), lambda i: (i, 0))],
        out_shape=[jax.ShapeDtypeStruct((NPAD, D), jnp.float32),
                   jax.ShapeDtypeStruct((NPAD, D), jnp.float32)],
    )(ae_g, tg_g, Wm0)


def _msg_body(beta, v4_ref, ee_ref, hs_ref, wr_ref, mn_ref, m8_ref):
    v4 = v4_ref[...]
    vx = v4[:, 0:1]
    vy = v4[:, 1:2]
    vz = v4[:, 2:3]
    mok = v4[:, 3:4]
    d2 = vx * vx + vy * vy + vz * vz
    dist = jnp.sqrt(d2 + 1e-12)
    inv = 1.0 / dist
    ed = jnp.exp(-dist)
    rbf = jnp.exp(-beta * (ed - mn_ref[...]) ** 2)
    fcut = 0.5 * (jnp.cos(jnp.float32(np.pi) / CUTOFF * dist) + 1.0)
    fcut = fcut * (dist < CUTOFF).astype(jnp.float32)
    filt = (rbf @ wr_ref[...]) * fcut + ee_ref[...]
    m = hs_ref[...] * filt * mok
    d0m = (vx * inv) * m
    d1m = (vy * inv) * m
    d2m = (vz * inv) * m
    chans = [m, d0m, d1m, d2m]
    m8_ref[...] = jnp.stack(
        [chans[ch][:, q * RACC:(q + 1) * RACC]
         for q in range(4) for ch in range(4)], axis=0)


def _tc_msg(V4, eE, hs, Wr, EPAD, BE=512):
    means = np.zeros((1, RPAD), np.float32)
    means[0, :R] = np.linspace(np.exp(-CUTOFF), 1.0, R)
    beta = np.float32(((2.0 / R) * (1.0 - np.exp(-CUTOFF))) ** -2)
    body = functools.partial(_msg_body, beta)
    return pl.pallas_call(
        body,
        grid=(EPAD // BE,),
        in_specs=[pl.BlockSpec((BE, 4), lambda i: (i, 0)),
                  pl.BlockSpec((BE, D), lambda i: (i, 0)),
                  pl.BlockSpec((BE, D), lambda i: (i, 0)),
                  pl.BlockSpec((RPAD, D), lambda i: (0, 0)),
                  pl.BlockSpec((1, RPAD), lambda i: (0, 0))],
        out_specs=pl.BlockSpec((16, BE, RACC), lambda i: (0, i, 0)),
        out_shape=jax.ShapeDtypeStruct((16, EPAD, RACC), jnp.float32),
    )(V4, eE, hs, Wr, jnp.asarray(means))


def _upd_body(has_x, has_hm, refs):
    i = 0
    ag_ref = refs[i]; i += 1
    h_ref = refs[i]; i += 1
    x_ref = None
    if has_x:
        x_ref = refs[i]; i += 1
    wu_ref = refs[i]; i += 1
    wv_ref = refs[i]; i += 1
    wm_ref = None
    if has_hm:
        wm_ref = refs[i]; i += 1
    ho_ref = refs[i]; i += 1
    xo_ref = refs[i]; i += 1
    hmo_ref = refs[i] if has_hm else None

    ag = ag_ref[...]
    agg = jnp.concatenate([ag[q * 4] for q in range(4)], axis=-1)
    h = h_ref[...] + jax.nn.silu(agg @ wu_ref[...])
    ho_ref[...] = h
    wv = wv_ref[...]
    ys = []
    for k in range(3):
        y = jnp.concatenate([ag[q * 4 + 1 + k] for q in range(4)],
                            axis=-1) @ wv
        ys.append(y)
    xl = jnp.stack(ys, axis=0)
    if has_x:
        xl = xl + x_ref[...]
    xo_ref[...] = xl
    if has_hm:
        hmo_ref[...] = jax.nn.silu(h @ wm_ref[...])


def _tc_upd(AG, h, X, Wu, Wv, Wm_next, NPAD, B=1024):
    has_x = X is not None
    has_hm = Wm_next is not None
    body = functools.partial(_upd_body, has_x, has_hm)

    def bwrap(*refs):
        body(refs)

    in_specs = [pl.BlockSpec((16, B, RACC), lambda i: (0, i, 0)),
                pl.BlockSpec((B, D), lambda i: (i, 0))]
    args = [AG, h]
    if has_x:
        in_specs.append(pl.BlockSpec((3, B, D), lambda i: (0, i, 0)))
        args.append(X)
    in_specs += [pl.BlockSpec((D, D), lambda i: (0, 0)),
                 pl.BlockSpec((D, D), lambda i: (0, 0))]
    args += [Wu, Wv]
    if has_hm:
        in_specs.append(pl.BlockSpec((D, D), lambda i: (0, 0)))
        args.append(Wm_next)
    out_specs = [pl.BlockSpec((B, D), lambda i: (i, 0)),
                 pl.BlockSpec((3, B, D), lambda i: (0, i, 0))]
    out_shape = [jax.ShapeDtypeStruct((NPAD, D), jnp.float32),
                 jax.ShapeDtypeStruct((3, NPAD, D), jnp.float32)]
    if has_hm:
        out_specs.append(pl.BlockSpec((B, D), lambda i: (i, 0)))
        out_shape.append(jax.ShapeDtypeStruct((NPAD, D), jnp.float32))
    return pl.pallas_call(
        bwrap,
        grid=(NPAD // B,),
        in_specs=in_specs,
        out_specs=out_specs,
        out_shape=out_shape,
    )(*args)


def _out_body(h_ref, x_ref, w1_ref, w2_ref, o_ref):
    s = jax.nn.silu(h_ref[...] @ w1_ref[...]) @ w2_ref[...]
    o_ref[...] = jnp.sum(x_ref[...] * s[None, :, :], axis=-1).T


def _tc_out(h, X3, W_out1, W_out2, NPAD, B=1024):
    return pl.pallas_call(
        _out_body,
        grid=(NPAD // B,),
        in_specs=[pl.BlockSpec((B, D), lambda i: (i, 0)),
                  pl.BlockSpec((3, B, D), lambda i: (0, i, 0)),
                  pl.BlockSpec((D, D), lambda i: (0, 0)),
                  pl.BlockSpec((D, D), lambda i: (0, 0))],
        out_specs=pl.BlockSpec((B, 3), lambda i: (i, 0)),
        out_shape=jax.ShapeDtypeStruct((NPAD, 3), jnp.float32),
    )(h, X3, W_out1, W_out2)


def kernel(ts_coord_t, numbers_t, bmat_r_t, bmat_p_t, edge_index, batch, time,
           atom_emb, edge_emb, W_time, W_rfeat, W_pfeat,
           W_msg, W_rbf, W_upd, W_vec, W_out1, W_out2):
    N = ts_coord_t.shape[0]
    E = edge_index.shape[1]
    n_ch = -(-N // (NW * CH))
    NPAD = NW * CH * n_ch
    e_ch = -(-E // (NW * CH))
    NB = 7
    e_ch = -(-e_ch // NB) * NB
    EPAD = NW * CH * e_ch

    f32 = jnp.float32
    numbers_p = jnp.pad(numbers_t, (0, NPAD - N))
    batch_p = jnp.pad(batch, (0, NPAD - N))
    batchf_p = batch_p.astype(f32)
    coordf_p = jnp.pad(ts_coord_t, ((0, NPAD - N), (0, 0))).reshape(-1)
    src_p = jnp.pad(edge_index[0], (0, EPAD - E))
    dst_p = jnp.pad(edge_index[1], (0, EPAD - E))
    bmr_p = jnp.pad(bmat_r_t, (0, EPAD - E))
    bmp_p = jnp.pad(bmat_p_t, (0, EPAD - E))
    src2 = src_p.reshape(EPAD // CH, CH)
    dst2 = dst_p.reshape(EPAD // CH, CH)
    T = time[:, None] * W_time
    Wr_pad = jnp.pad(W_rbf, ((0, 0), (0, RPAD - R), (0, 0)))
    zrow = jnp.zeros((CH, RACC), f32)

    ae_g, tg_g, Pflat = _k_nodeprep(NPAD, n_ch)(
        numbers_p, batch_p, coordf_p, batchf_p, atom_emb, T)
    V4f, eE = _k_edgeprep(NPAD, EPAD, e_ch)(
        src_p, dst_p, bmr_p, bmp_p, Pflat.reshape(NPAD, PW), edge_emb)
    V4 = V4f.reshape(EPAD, 4)

    h, hm = _tc_h0(ae_g, tg_g, W_msg[0], NPAD)
    X = None
    kg = _k_rowgather(NPAD, EPAD, e_ch, NB)
    ks = _k_scatter(NPAD, EPAD, e_ch, NB)
    for l in range(L):
        hs = kg(hm, src2)
        M8 = _tc_msg(V4, eE, hs, Wr_pad[l], EPAD)
        AG = ks(M8, dst2, zrow)
        Wm_next = W_msg[l + 1] if l + 1 < L else None
        res = _tc_upd(AG, h, X, W_upd[l], W_vec[l], Wm_next, NPAD)
        if Wm_next is not None:
            h, X, hm = res
        else:
            h, X = res
    return _tc_out(h, X, W_out1, W_out2, NPAD)[:N]


# speedup vs baseline: 13.5722x; 1.5307x over previous
import functools

import jax
import jax.numpy as jnp
import numpy as np
from jax import lax
from jax.experimental import pallas as pl
from jax.experimental.pallas import tpu as pltpu
from jax.experimental.pallas import tpu_sc as plsc

NC, NS = 2, 16
NW = NC * NS
CH = 128
D = 64
RPAD = 32
RACC = 16
PW = 16
CUTOFF = 5.0
R = 20
L = 3

_MESH = dict(core_axis_name="c", subcore_axis_name="s")


def _it16():
    return lax.iota(jnp.int32, 16)


def _wid():
    return lax.axis_index("s") * NC + lax.axis_index("c")


def _k_nodeprep(NPAD, n_ch):
    npw = n_ch * CH

    def body(num_ref, bat_ref, coordf_ref, batchf_ref, ae_ref, t_ref,
             aeg_ref, tgg_ref, p_ref, nidx, bidx, rows_a, rows_t, cbuf, bbuf,
             pbuf, sem):
        wid = _wid()

        def chunk(ci, carry):
            base = wid * npw + ci * CH
            pltpu.sync_copy(num_ref.at[pl.ds(base, CH)], nidx)
            pltpu.async_copy(ae_ref.at[nidx], rows_a, sem).wait()
            pltpu.sync_copy(rows_a, aeg_ref.at[pl.ds(base, CH), :])
            pltpu.sync_copy(bat_ref.at[pl.ds(base, CH)], bidx)
            pltpu.async_copy(t_ref.at[bidx], rows_t, sem).wait()
            pltpu.sync_copy(rows_t, tgg_ref.at[pl.ds(base, CH), :])
            pltpu.sync_copy(coordf_ref.at[pl.ds(base * 3, CH * 3)], cbuf)
            pltpu.sync_copy(batchf_ref.at[pl.ds(base, CH)], bbuf)
            it = _it16()
            for g in range(CH // 16):
                j16 = g * 16 + it
                for f in range(3):
                    vals = plsc.load_gather(cbuf, [j16 * 3 + f])
                    plsc.store_scatter(pbuf, [j16 * PW + f], vals)
                bv = bbuf[pl.ds(g * 16, 16)]
                plsc.store_scatter(pbuf, [j16 * PW + 3], bv)
            pltpu.sync_copy(pbuf, p_ref.at[pl.ds(base * PW, CH * PW)])
            return carry

        lax.fori_loop(0, n_ch, chunk, 0)

    return pl.kernel(
        body,
        out_type=(jax.ShapeDtypeStruct((NPAD, D), jnp.float32),
                  jax.ShapeDtypeStruct((NPAD, D), jnp.float32),
                  jax.ShapeDtypeStruct((NPAD * PW,), jnp.float32)),
        mesh=plsc.VectorSubcoreMesh(**_MESH),
        compiler_params=pltpu.CompilerParams(needs_layout_passes=False, use_tc_tiling_on_sc=False),
        scratch_types=[pltpu.VMEM((CH,), jnp.int32),
                       pltpu.VMEM((CH,), jnp.int32),
                       pltpu.VMEM((CH, D), jnp.float32),
                       pltpu.VMEM((CH, D), jnp.float32),
                       pltpu.VMEM((CH * 3,), jnp.float32),
                       pltpu.VMEM((CH,), jnp.float32),
                       pltpu.VMEM((CH * PW,), jnp.float32),
                       pltpu.SemaphoreType.DMA])


def _k_edgeprep(NPAD, EPAD, e_ch):
    epw = e_ch * CH

    def body(src_ref, dst_ref, bmr_ref, bmp_ref, p_ref, ee_ref,
             v4_ref, eeg_ref, sidx, didx, rb, pb2, etb, ps, pd, vbuf, erows,
             sem):
        wid = _wid()

        def chunk(ci, carry):
            base = wid * epw + ci * CH
            pltpu.sync_copy(src_ref.at[pl.ds(base, CH)], sidx)
            pltpu.sync_copy(dst_ref.at[pl.ds(base, CH)], didx)
            pltpu.sync_copy(bmr_ref.at[pl.ds(base, CH)], rb)
            pltpu.sync_copy(bmp_ref.at[pl.ds(base, CH)], pb2)
            pltpu.async_copy(p_ref.at[sidx], ps, sem).wait()
            pltpu.async_copy(p_ref.at[didx], pd, sem).wait()
            for g in range(CH // 16):
                sl = pl.ds(g * 16, 16)
                etb[sl] = rb[sl] * 22 + pb2[sl]
            pltpu.async_copy(ee_ref.at[etb], erows, sem).wait()
            pltpu.sync_copy(erows, eeg_ref.at[pl.ds(base, CH), :])
            it = _it16()
            for g in range(CH // 16):
                j16 = g * 16 + it
                for f in range(3):
                    fv = jnp.full((16,), f, jnp.int32)
                    dvf = plsc.load_gather(pd, [j16, fv])
                    svf = plsc.load_gather(ps, [j16, fv])
                    plsc.store_scatter(vbuf, [j16 * 4 + f], dvf - svf)
                c3 = jnp.full((16,), 3, jnp.int32)
                bs = plsc.load_gather(ps, [j16, c3])
                bd = plsc.load_gather(pd, [j16, c3])
                sl = pl.ds(g * 16, 16)
                sv = sidx[sl]
                dv = didx[sl]
                ev = etb[sl]
                mok = jnp.where((ev != 0) & (sv != dv) & (bs == bd),
                                jnp.float32(1.0), jnp.float32(0.0))
                plsc.store_scatter(vbuf, [j16 * 4 + 3], mok)
            pltpu.sync_copy(vbuf, v4_ref.at[pl.ds(base * 4, CH * 4)])
            return carry

        lax.fori_loop(0, e_ch, chunk, 0)

    return pl.kernel(
        body,
        out_type=(jax.ShapeDtypeStruct((EPAD * 4,), jnp.float32),
                  jax.ShapeDtypeStruct((EPAD, D), jnp.float32)),
        mesh=plsc.VectorSubcoreMesh(**_MESH),
        compiler_params=pltpu.CompilerParams(needs_layout_passes=False, use_tc_tiling_on_sc=False),
        scratch_types=[pltpu.VMEM((CH,), jnp.int32),
                       pltpu.VMEM((CH,), jnp.int32),
                       pltpu.VMEM((CH,), jnp.int32),
                       pltpu.VMEM((CH,), jnp.int32),
                       pltpu.VMEM((CH,), jnp.int32),
                       pltpu.VMEM((CH, PW), jnp.float32),
                       pltpu.VMEM((CH, PW), jnp.float32),
                       pltpu.VMEM((CH * 4,), jnp.float32),
                       pltpu.VMEM((CH, D), jnp.float32),
                       pltpu.SemaphoreType.DMA])


def _k_rowgather(NPAD, EPAD, BCG=448):
    epw = EPAD // NW
    nch = epw // BCG

    def body(tab_ref, srcf_ref, out_ref, ib0, ib1, r0b, r1b, gsem, wsem):
        wid = _wid()
        base = wid * epw
        banks = (r0b, r1b)
        ibs = (ib0, ib1)

        def grp(g2, carry):
            descs = []
            for par in range(2):
                g = g2 * 2 + par
                bank = banks[par]

                @pl.when(g2 > 0)
                def _():
                    pltpu.make_async_copy(tab_ref.at[pl.ds(0, BCG), :], bank,
                                          wsem).wait()

                pltpu.sync_copy(srcf_ref.at[pl.ds(base + g * BCG, BCG)],
                                ibs[par])
                descs.append(pltpu.async_copy(tab_ref.at[ibs[par]], bank,
                                              gsem))
            for par in range(2):
                g = g2 * 2 + par
                descs[par].wait()
                pltpu.async_copy(banks[par],
                                 out_ref.at[pl.ds(base + g * BCG, BCG), :],
                                 wsem)
            return carry

        lax.fori_loop(0, nch // 2, grp, 0)
        pltpu.make_async_copy(tab_ref.at[pl.ds(0, BCG), :], r0b, wsem).wait()
        pltpu.make_async_copy(tab_ref.at[pl.ds(0, BCG), :], r1b, wsem).wait()

    return pl.kernel(
        body,
        out_type=jax.ShapeDtypeStruct((EPAD, D), jnp.float32),
        mesh=plsc.VectorSubcoreMesh(**_MESH),
        compiler_params=pltpu.CompilerParams(needs_layout_passes=False, use_tc_tiling_on_sc=False),
        scratch_types=[pltpu.VMEM((BCG,), jnp.int32),
                       pltpu.VMEM((BCG,), jnp.int32),
                       pltpu.VMEM((BCG, D), jnp.float32),
                       pltpu.VMEM((BCG, D), jnp.float32),
                       pltpu.SemaphoreType.DMA,
                       pltpu.SemaphoreType.DMA])


def _k_scatter(NPAD, EPAD, NB=7):
    BC = NB * CH
    epw = EPAD // NS
    nch = epw // BC
    npw_s = NPAD // NS
    nzch = npw_s // CH

    def body(m8_ref, dstf_ref, zrow_ref, ag_ref, acc, zbuf, bank0, bank1,
             ib0, ib1, ssem):
        c = lax.axis_index("c")
        s = lax.axis_index("s")
        banks = (bank0, bank1)
        ibs = (ib0, ib1)
        pltpu.sync_copy(zrow_ref, zbuf)
        ebase = s * epw
        for cc_l in range(8):
            half, ch = divmod(cc_l, 4)
            cc = (c * 2 + half) * 4 + ch

            def zc(z, carry):
                pltpu.sync_copy(zbuf, acc.at[pl.ds(s * npw_s + z * CH, CH), :])
                return carry

            lax.fori_loop(0, nzch, zc, 0)
            plsc.subcore_barrier()

            def grp(g2, carry):
                for par in range(2):
                    g = g2 * 2 + par
                    bank = banks[par]
                    ib = ibs[par]

                    co = (half * 4 + ch) * RACC

                    @pl.when(g2 > 0)
                    def _():
                        pltpu.make_async_copy(
                            m8_ref.at[c, pl.ds(0, BC), pl.ds(co, RACC)],
                            bank, ssem).wait()

                    e0 = ebase + g * BC
                    pltpu.sync_copy(dstf_ref.at[pl.ds(e0, BC)], ib)
                    pltpu.sync_copy(m8_ref.at[c, pl.ds(e0, BC), pl.ds(co, RACC)],
                                    bank)
                    pltpu.async_copy(bank, acc.at[ib], ssem, add=True)
                return carry

            lax.fori_loop(0, nch // 2, grp, 0)
            co2 = (half * 4 + ch) * RACC
            pltpu.make_async_copy(m8_ref.at[c, pl.ds(0, BC), pl.ds(co2, RACC)],
                                  bank0, ssem).wait()
            pltpu.make_async_copy(m8_ref.at[c, pl.ds(0, BC), pl.ds(co2, RACC)],
                                  bank1, ssem).wait()
            plsc.subcore_barrier()

            def wb(z, carry):
                r0 = s * npw_s + z * CH
                pltpu.sync_copy(acc.at[pl.ds(r0, CH), :],
                                ag_ref.at[cc, pl.ds(r0, CH), :])
                return carry

            lax.fori_loop(0, nzch, wb, 0)
            plsc.subcore_barrier()

    return pl.kernel(
        body,
        out_type=jax.ShapeDtypeStruct((16, NPAD, RACC), jnp.float32),
        mesh=plsc.VectorSubcoreMesh(**_MESH),
        compiler_params=pltpu.CompilerParams(needs_layout_passes=False, use_tc_tiling_on_sc=False),
        scratch_types=[pltpu.VMEM_SHARED((NPAD, RACC), jnp.float32),
                       pltpu.VMEM((CH, RACC), jnp.float32),
                       pltpu.VMEM((BC, RACC), jnp.float32),
                       pltpu.VMEM((BC, RACC), jnp.float32),
                       pltpu.VMEM((BC,), jnp.int32),
                       pltpu.VMEM((BC,), jnp.int32),
                       pltpu.SemaphoreType.DMA])


def _h0_body(ae_ref, tg_ref, wm_ref, h_ref, hm_ref):
    h = ae_ref[...] + tg_ref[...]
    h_ref[...] = h
    hm_ref[...] = jax.nn.silu(h @ wm_ref[...])


def _tc_h0(ae_g, tg_g, Wm0, NPAD, B=1024):
    return pl.pallas_call(
        _h0_body,
        grid=(NPAD // B,),
        in_specs=[pl.BlockSpec((B, D), lambda i: (i, 0)),
                  pl.BlockSpec((B, D), lambda i: (i, 0)),
                  pl.BlockSpec((D, D), lambda i: (0, 0))],
        out_specs=[pl.BlockSpec((B, D), lambda i: (i, 0)),
                   pl.BlockSpec((B, D), lambda i: (i, 0))],
        out_shape=[jax.ShapeDtypeStruct((NPAD, D), jnp.float32),
                   jax.ShapeDtypeStruct((NPAD, D), jnp.float32)],
    )(ae_g, tg_g, Wm0)


def _msg_body(beta, v4_ref, ee_ref, hs_ref, wr_ref, mn_ref, m8_ref):
    v4 = v4_ref[...]
    vx = v4[:, 0:1]
    vy = v4[:, 1:2]
    vz = v4[:, 2:3]
    mok = v4[:, 3:4]
    d2 = vx * vx + vy * vy + vz * vz
    dist = jnp.sqrt(d2 + 1e-12)
    inv = 1.0 / dist
    ed = jnp.exp(-dist)
    rbf = jnp.exp(-beta * (ed - mn_ref[...]) ** 2)
    fcut = 0.5 * (jnp.cos(jnp.float32(np.pi) / CUTOFF * dist) + 1.0)
    fcut = fcut * (dist < CUTOFF).astype(jnp.float32)
    filt = (rbf @ wr_ref[...]) * fcut + ee_ref[...]
    m = hs_ref[...] * filt * mok
    d0m = (vx * inv) * m
    d1m = (vy * inv) * m
    d2m = (vz * inv) * m
    chans = [m, d0m, d1m, d2m]
    planes = []
    for w in range(2):
        parts = [chans[ch][:, (2 * w + qq) * RACC:(2 * w + qq + 1) * RACC]
                 for qq in range(2) for ch in range(4)]
        planes.append(jnp.concatenate(parts, axis=-1))
    m8_ref[...] = jnp.stack(planes, axis=0)


def _tc_msg(V4, eE, hs, Wr, EPAD, BE=512):
    means = np.zeros((1, RPAD), np.float32)
    means[0, :R] = np.linspace(np.exp(-CUTOFF), 1.0, R)
    beta = np.float32(((2.0 / R) * (1.0 - np.exp(-CUTOFF))) ** -2)
    body = functools.partial(_msg_body, beta)
    return pl.pallas_call(
        body,
        grid=(EPAD // BE,),
        in_specs=[pl.BlockSpec((BE, 4), lambda i: (i, 0)),
                  pl.BlockSpec((BE, D), lambda i: (i, 0)),
                  pl.BlockSpec((BE, D), lambda i: (i, 0)),
                  pl.BlockSpec((RPAD, D), lambda i: (0, 0)),
                  pl.BlockSpec((1, RPAD), lambda i: (0, 0))],
        out_specs=pl.BlockSpec((2, BE, 8 * RACC), lambda i: (0, i, 0)),
        out_shape=jax.ShapeDtypeStruct((2, EPAD, 8 * RACC), jnp.float32),
    )(V4, eE, hs, Wr, jnp.asarray(means))


def _upd_body(has_x, has_hm, refs):
    i = 0
    ag_ref = refs[i]; i += 1
    h_ref = refs[i]; i += 1
    x_ref = None
    if has_x:
        x_ref = refs[i]; i += 1
    wu_ref = refs[i]; i += 1
    wv_ref = refs[i]; i += 1
    wm_ref = None
    if has_hm:
        wm_ref = refs[i]; i += 1
    ho_ref = refs[i]; i += 1
    xo_ref = refs[i]; i += 1
    hmo_ref = refs[i] if has_hm else None

    ag = ag_ref[...]
    agg = jnp.concatenate([ag[q * 4] for q in range(4)], axis=-1)
    h = h_ref[...] + jax.nn.silu(agg @ wu_ref[...])
    ho_ref[...] = h
    wv = wv_ref[...]
    ys = []
    for k in range(3):
        y = jnp.concatenate([ag[q * 4 + 1 + k] for q in range(4)],
                            axis=-1) @ wv
        ys.append(y)
    xl = jnp.stack(ys, axis=0)
    if has_x:
        xl = xl + x_ref[...]
    xo_ref[...] = xl
    if has_hm:
        hmo_ref[...] = jax.nn.silu(h @ wm_ref[...])


def _tc_upd(AG, h, X, Wu, Wv, Wm_next, NPAD, B=1024):
    has_x = X is not None
    has_hm = Wm_next is not None
    body = functools.partial(_upd_body, has_x, has_hm)

    def bwrap(*refs):
        body(refs)

    in_specs = [pl.BlockSpec((16, B, RACC), lambda i: (0, i, 0)),
                pl.BlockSpec((B, D), lambda i: (i, 0))]
    args = [AG, h]
    if has_x:
        in_specs.append(pl.BlockSpec((3, B, D), lambda i: (0, i, 0)))
        args.append(X)
    in_specs += [pl.BlockSpec((D, D), lambda i: (0, 0)),
                 pl.BlockSpec((D, D), lambda i: (0, 0))]
    args += [Wu, Wv]
    if has_hm:
        in_specs.append(pl.BlockSpec((D, D), lambda i: (0, 0)))
        args.append(Wm_next)
    out_specs = [pl.BlockSpec((B, D), lambda i: (i, 0)),
                 pl.BlockSpec((3, B, D), lambda i: (0, i, 0))]
    out_shape = [jax.ShapeDtypeStruct((NPAD, D), jnp.float32),
                 jax.ShapeDtypeStruct((3, NPAD, D), jnp.float32)]
    if has_hm:
        out_specs.append(pl.BlockSpec((B, D), lambda i: (i, 0)))
        out_shape.append(jax.ShapeDtypeStruct((NPAD, D), jnp.float32))
    return pl.pallas_call(
        bwrap,
        grid=(NPAD // B,),
        in_specs=in_specs,
        out_specs=out_specs,
        out_shape=out_shape,
    )(*args)


def _out_body(h_ref, x_ref, w1_ref, w2_ref, o_ref):
    s = jax.nn.silu(h_ref[...] @ w1_ref[...]) @ w2_ref[...]
    o_ref[...] = jnp.sum(x_ref[...] * s[None, :, :], axis=-1).T


def _tc_out(h, X3, W_out1, W_out2, NPAD, B=1024):
    return pl.pallas_call(
        _out_body,
        grid=(NPAD // B,),
        in_specs=[pl.BlockSpec((B, D), lambda i: (i, 0)),
                  pl.BlockSpec((3, B, D), lambda i: (0, i, 0)),
                  pl.BlockSpec((D, D), lambda i: (0, 0)),
                  pl.BlockSpec((D, D), lambda i: (0, 0))],
        out_specs=pl.BlockSpec((B, 3), lambda i: (i, 0)),
        out_shape=jax.ShapeDtypeStruct((NPAD, 3), jnp.float32),
    )(h, X3, W_out1, W_out2)


def kernel(ts_coord_t, numbers_t, bmat_r_t, bmat_p_t, edge_index, batch, time,
           atom_emb, edge_emb, W_time, W_rfeat, W_pfeat,
           W_msg, W_rbf, W_upd, W_vec, W_out1, W_out2):
    N = ts_coord_t.shape[0]
    E = edge_index.shape[1]
    n_ch = -(-N // (NW * CH))
    NPAD = NW * CH * n_ch
    e_ch = -(-E // (NW * CH))
    NB = 7
    e_ch = -(-e_ch // NB) * NB
    EPAD = NW * CH * e_ch

    f32 = jnp.float32
    numbers_p = jnp.pad(numbers_t, (0, NPAD - N))
    batch_p = jnp.pad(batch, (0, NPAD - N))
    batchf_p = batch_p.astype(f32)
    coordf_p = jnp.pad(ts_coord_t, ((0, NPAD - N), (0, 0))).reshape(-1)
    src_p = jnp.pad(edge_index[0], (0, EPAD - E))
    dst_p = jnp.pad(edge_index[1], (0, EPAD - E))
    bmr_p = jnp.pad(bmat_r_t, (0, EPAD - E))
    bmp_p = jnp.pad(bmat_p_t, (0, EPAD - E))
    T = time[:, None] * W_time
    Wr_pad = jnp.pad(W_rbf, ((0, 0), (0, RPAD - R), (0, 0)))
    zrow = jnp.zeros((CH, RACC), f32)

    ae_g, tg_g, Pflat = _k_nodeprep(NPAD, n_ch)(
        numbers_p, batch_p, coordf_p, batchf_p, atom_emb, T)
    V4f, eE = _k_edgeprep(NPAD, EPAD, e_ch)(
        src_p, dst_p, bmr_p, bmp_p, Pflat.reshape(NPAD, PW), edge_emb)
    V4 = V4f.reshape(EPAD, 4)

    h, hm = _tc_h0(ae_g, tg_g, W_msg[0], NPAD)
    X = None
    kg = _k_rowgather(NPAD, EPAD)
    ks = _k_scatter(NPAD, EPAD)
    for l in range(L):
        hs = kg(hm, src_p)
        M8 = _tc_msg(V4, eE, hs, Wr_pad[l], EPAD)
        AG = ks(M8, dst_p, zrow)
        Wm_next = W_msg[l + 1] if l + 1 < L else None
        res = _tc_upd(AG, h, X, W_upd[l], W_vec[l], Wm_next, NPAD)
        if Wm_next is not None:
            h, X, hm = res
        else:
            h, X = res
    return _tc_out(h, X, W_out1, W_out2, NPAD)[:N]
